# baseline (speedup 1.0000x reference)
import jax
import jax.numpy as jnp
from jax import lax
from jax.experimental import pallas as pl
from jax.experimental.pallas import tpu as pltpu
from jax.experimental.pallas import tpu_sc as plsc

_D = _H = _W = 64
_EMB = 64
_ROW = 3 * _EMB
_N = _D * _H * _W
_NV = _EMB // 16
_NBUF = 4
_TCG = 32
_TCSEM = 8


def _sc_body(d_hbm, h_hbm, out_hbm, tab_d, tab_h, *rest):
    blks = rest[:_NBUF]
    sems = rest[_NBUF:]
    wid = lax.axis_index("s") * 2 + lax.axis_index("c")

    pltpu.sync_copy(d_hbm.at[pl.ds(0, _D)], tab_d)
    pltpu.sync_copy(h_hbm.at[pl.ds(0, _H)], tab_h)

    def fill_h(h, blk):
        hv = [tab_h[h, pl.ds(16 * k, 16)] for k in range(_NV)]
        def body(r, carry):
            for k in range(_NV):
                blk[r, pl.ds(_EMB + 16 * k, 16)] = hv[k]
            return carry
        lax.fori_loop(0, _W, body, 0)

    for dd in range(2):
        d = wid * 2 + dd
        dv = [tab_d[d, pl.ds(16 * k, 16)] for k in range(_NV)]

        def fill_d(r, carry):
            for k in range(_NV):
                for blk in blks:
                    blk[r, pl.ds(16 * k, 16)] = dv[k]
            return carry
        lax.fori_loop(0, _W, fill_d, 0)

        base = d * (_H * _W)

        for p in range(_NBUF):
            fill_h(p, blks[p])
            pltpu.async_copy(
                blks[p], out_hbm.at[0, pl.ds(base + p * _W, _W),
                                    pl.ds(0, 2 * _EMB)], sems[p])

        def pipe(i, carry):
            for p in range(_NBUF):
                h = i * _NBUF + p
                pltpu.make_async_copy(
                    blks[p], out_hbm.at[0, pl.ds(base, _W),
                                        pl.ds(0, 2 * _EMB)],
                    sems[p]).wait()
                fill_h(h, blks[p])
                pltpu.async_copy(
                    blks[p], out_hbm.at[0, pl.ds(base + h * _W, _W),
                                        pl.ds(0, 2 * _EMB)], sems[p])
            return carry
        lax.fori_loop(1, _H // _NBUF, pipe, 0)

        for p in range(_NBUF):
            pltpu.make_async_copy(
                blks[p], out_hbm.at[0, pl.ds(base, _W),
                                    pl.ds(0, 2 * _EMB)], sems[p]).wait()


def _tc_body(w_ref, part_ref, out_ref):
    rows = _N // _TCG
    rep = rows // _W
    out_ref[:, :, : 2 * _EMB] = part_ref[...]
    out_ref[:, :, 2 * _EMB:] = jnp.broadcast_to(
        w_ref[pl.ds(0, _W), :][None, :, :], (rep, _W, _EMB)
    ).reshape(1, rows, _EMB)


def kernel(depth, height, width, batch_size, d_emb, h_emb, w_emb):
    mesh = plsc.VectorSubcoreMesh(core_axis_name="c", subcore_axis_name="s")
    part = pl.kernel(
        _sc_body,
        out_type=jax.ShapeDtypeStruct((1, _N, _ROW), jnp.float32),
        mesh=mesh,
        compiler_params=pltpu.CompilerParams(use_tc_tiling_on_sc=True),
        scratch_types=(
            [pltpu.VMEM((_D, _EMB), jnp.float32)] * 2
            + [pltpu.VMEM((_W, 2 * _EMB), jnp.float32)] * _NBUF
            + [pltpu.SemaphoreType.DMA] * _NBUF
        ),
    )(d_emb, h_emb)

    rows = _N // _TCG
    out = pl.pallas_call(
        _tc_body,
        out_shape=jax.ShapeDtypeStruct((1, _N, _ROW), jnp.float32),
        grid=(_TCG,),
        in_specs=[
            pl.BlockSpec((128, _EMB), lambda i: (0, 0)),
            pl.BlockSpec((1, rows, 2 * _EMB), lambda i: (0, i, 0)),
        ],
        out_specs=pl.BlockSpec((1, rows, _ROW), lambda i: (0, i, 0)),
        input_output_aliases={1: 0},
    )(w_emb, part)
    return out


# speedup vs baseline: 1.4225x; 1.4225x over previous
import jax
import jax.numpy as jnp
from jax import lax
from jax.experimental import pallas as pl
from jax.experimental.pallas import tpu as pltpu
from jax.experimental.pallas import tpu_sc as plsc

_D = _H = _W = 64
_EMB = 64
_ROW = 3 * _EMB
_NV = _EMB // 16
_NBUF = 6


def _body(d_hbm, h_hbm, w_hbm, out_hbm, tab_d, tab_h, tab_w, *rest):
    blks = rest[:_NBUF]
    sems = rest[_NBUF:]
    wid = lax.axis_index("s") * 2 + lax.axis_index("c")

    pltpu.sync_copy(d_hbm.at[pl.ds(0, _D)], tab_d)
    pltpu.sync_copy(h_hbm.at[pl.ds(0, _H)], tab_h)
    pltpu.sync_copy(w_hbm.at[pl.ds(0, _W)], tab_w)

    def fill_w(r, carry):
        for k in range(_NV):
            v = tab_w[r, pl.ds(16 * k, 16)]
            for blk in blks:
                blk[r, pl.ds(2 * _EMB + 16 * k, 16)] = v
        return carry
    lax.fori_loop(0, _W, fill_w, 0)

    def fill_h(h, blk):
        hv = [tab_h[h, pl.ds(16 * k, 16)] for k in range(_NV)]
        def body(r, carry):
            for k in range(_NV):
                blk[r, pl.ds(_EMB + 16 * k, 16)] = hv[k]
            return carry
        lax.fori_loop(0, _W, body, 0)

    for dd in range(2):
        d = wid * 2 + dd
        dv = [tab_d[d, pl.ds(16 * k, 16)] for k in range(_NV)]

        def fill_d(r, carry):
            for k in range(_NV):
                for blk in blks:
                    blk[r, pl.ds(16 * k, 16)] = dv[k]
            return carry
        lax.fori_loop(0, _W, fill_d, 0)

        base = d * (_H * _W)

        for p in range(_NBUF):
            fill_h(p, blks[p])
            pltpu.async_copy(
                blks[p], out_hbm.at[pl.ds(base + p * _W, _W)], sems[p])

        def step(h, p):
            pltpu.make_async_copy(
                blks[p], out_hbm.at[pl.ds(base, _W)], sems[p]).wait()
            fill_h(h, blks[p])
            pltpu.async_copy(
                blks[p], out_hbm.at[pl.ds(base + h * _W, _W)], sems[p])

        def pipe(i, carry):
            for p in range(_NBUF):
                step(i * _NBUF + p, p)
            return carry
        nround = _H // _NBUF
        lax.fori_loop(1, nround, pipe, 0)
        for t in range(nround * _NBUF, _H):
            step(t, t - nround * _NBUF)

        for p in range(_NBUF):
            pltpu.make_async_copy(
                blks[p], out_hbm.at[pl.ds(base, _W)], sems[p]).wait()


def kernel(depth, height, width, batch_size, d_emb, h_emb, w_emb):
    mesh = plsc.VectorSubcoreMesh(core_axis_name="c", subcore_axis_name="s")
    out = pl.kernel(
        _body,
        out_type=jax.ShapeDtypeStruct((_D * _H * _W, _ROW), jnp.float32),
        mesh=mesh,
        scratch_types=(
            [pltpu.VMEM((_D, _EMB), jnp.float32)] * 3
            + [pltpu.VMEM((_W, _ROW), jnp.float32)] * _NBUF
            + [pltpu.SemaphoreType.DMA] * _NBUF
        ),
    )(d_emb, h_emb, w_emb)
    return out.reshape(1, _D * _H * _W, _ROW)


# speedup vs baseline: 1.4377x; 1.0107x over previous
import jax
import jax.numpy as jnp
from jax import lax
from jax.experimental import pallas as pl
from jax.experimental.pallas import tpu as pltpu
from jax.experimental.pallas import tpu_sc as plsc

_D = _H = _W = 64
_EMB = 64
_ROW = 3 * _EMB
_NV = _EMB // 16
_NBUF = 2


def _body(d_hbm, h_hbm, w_hbm, out_hbm, tab_d, tab_h, tab_w, *rest):
    blks = rest[:_NBUF]
    sems = rest[_NBUF:]
    wid = lax.axis_index("s") * 2 + lax.axis_index("c")

    pltpu.sync_copy(d_hbm.at[pl.ds(0, _D)], tab_d)
    pltpu.sync_copy(h_hbm.at[pl.ds(0, _H)], tab_h)
    pltpu.sync_copy(w_hbm.at[pl.ds(0, _W)], tab_w)

    def fill_w(r, carry):
        for k in range(_NV):
            v = tab_w[r, pl.ds(16 * k, 16)]
            for blk in blks:
                blk[r, pl.ds(2 * _EMB + 16 * k, 16)] = v
        return carry
    lax.fori_loop(0, _W, fill_w, 0)

    def fill_h(h, blk):
        hv = [tab_h[h, pl.ds(16 * k, 16)] for k in range(_NV)]
        def body(r, carry):
            for k in range(_NV):
                blk[r, pl.ds(_EMB + 16 * k, 16)] = hv[k]
            return carry
        lax.fori_loop(0, _W, body, 0)

    for dd in range(2):
        d = wid * 2 + dd
        dv = [tab_d[d, pl.ds(16 * k, 16)] for k in range(_NV)]

        def fill_d(r, carry):
            for k in range(_NV):
                for blk in blks:
                    blk[r, pl.ds(16 * k, 16)] = dv[k]
            return carry
        lax.fori_loop(0, _W, fill_d, 0)

        base = d * (_H * _W)

        for p in range(_NBUF):
            fill_h(p, blks[p])
            pltpu.async_copy(
                blks[p], out_hbm.at[pl.ds(base + p * _W, _W)], sems[p])

        def step(h, p):
            pltpu.make_async_copy(
                blks[p], out_hbm.at[pl.ds(base, _W)], sems[p]).wait()
            fill_h(h, blks[p])
            pltpu.async_copy(
                blks[p], out_hbm.at[pl.ds(base + h * _W, _W)], sems[p])

        def pipe(i, carry):
            for p in range(_NBUF):
                step(i * _NBUF + p, p)
            return carry
        nround = _H // _NBUF
        lax.fori_loop(1, nround, pipe, 0)
        for t in range(nround * _NBUF, _H):
            step(t, t - nround * _NBUF)

        for p in range(_NBUF):
            pltpu.make_async_copy(
                blks[p], out_hbm.at[pl.ds(base, _W)], sems[p]).wait()


def kernel(depth, height, width, batch_size, d_emb, h_emb, w_emb):
    mesh = plsc.VectorSubcoreMesh(core_axis_name="c", subcore_axis_name="s")
    out = pl.kernel(
        _body,
        out_type=jax.ShapeDtypeStruct((_D * _H * _W, _ROW), jnp.float32),
        mesh=mesh,
        scratch_types=(
            [pltpu.VMEM((_D, _EMB), jnp.float32)] * 3
            + [pltpu.VMEM((_W, _ROW), jnp.float32)] * _NBUF
            + [pltpu.SemaphoreType.DMA] * _NBUF
        ),
    )(d_emb, h_emb, w_emb)
    return out.reshape(1, _D * _H * _W, _ROW)
